# baseline (speedup 1.0000x reference)
import math

import jax
import jax.numpy as jnp
from jax.experimental import pallas as pl
from jax.experimental.pallas import tpu as pltpu

_V = 100000
_B = 1024
_SMOOTH = 0.1 / (_V - 2)
_CONF = 0.9
_ENT = (_V - 2) * _SMOOTH * math.log(_SMOOTH) + _CONF * math.log(_CONF)
_BLK = 2048
_GRID = (_V + _BLK - 1) // _BLK
_NCH = _BLK // 128


def _body(tgt_ref, out_ref, loss_ref, sacc_ref, c0_ref):
    j = pl.program_id(0)
    d = out_ref[...]
    t = tgt_ref[...]
    m = (t != 0)
    col = jax.lax.broadcasted_iota(jnp.int32, (_B, _BLK), 1)
    tl = t - j * _BLK

    @pl.when(j == 0)
    def _():
        c0_ref[...] = jnp.sum(jnp.where(m, d[:, 0:1], 0.0)).reshape(1, 1)
        sacc_ref[...] = jnp.zeros((1, 1), jnp.float32)

    wd = jnp.where(col == tl, _CONF, _SMOOTH) * d
    wd = jnp.where(m, wd, 0.0)

    @pl.when(j < _GRID - 1)
    def _():
        sacc_ref[...] += jnp.sum(wd)

    @pl.when(j == _GRID - 1)
    def _():
        s_wd = sacc_ref[...] + jnp.sum(jnp.where(col + j * _BLK < _V, wd, 0.0))
        n_valid = jnp.sum(jnp.where(m, 1.0, 0.0))
        loss_ref[...] = (_ENT * n_valid - s_wd + _SMOOTH * c0_ref[...])


def kernel(output, target):
    t2 = target.reshape(_B, 1)
    acc = pl.pallas_call(
        _body,
        grid=(_GRID,),
        in_specs=[
            pl.BlockSpec((_B, 1), lambda j: (0, 0)),
            pl.BlockSpec((_B, _BLK), lambda j: (0, j)),
        ],
        out_specs=pl.BlockSpec((1, 1), lambda j: (0, 0)),
        out_shape=jax.ShapeDtypeStruct((1, 1), jnp.float32),
        scratch_shapes=[
            pltpu.VMEM((1, 1), jnp.float32),
            pltpu.VMEM((1, 1), jnp.float32),
        ],
    )(t2, output)
    return acc[0, 0]
